# baseline (speedup 1.0000x reference)
import functools

import jax
import jax.numpy as jnp
from jax import lax
from jax.experimental import pallas as pl
from jax.experimental.pallas import tpu as pltpu

M = 32
K = 256
D = 4
F = 128
B = 16384
BB = 128


def _pq_body(x_ref, ct_ref, ck_ref, xhat_ref, codes_ref):
    x = x_ref[...]
    r = lax.broadcasted_iota(jnp.int32, (F, F), 0) // D
    c = lax.broadcasted_iota(jnp.int32, (F, F), 1) // D
    g = (r == c).astype(jnp.float32)
    ssq = jnp.dot(x * x, g, preferred_element_type=jnp.float32,
                  precision=lax.Precision.HIGHEST)
    inv = lax.rsqrt(jnp.maximum(ssq, 1e-24))
    xn = x * inv
    for m in range(M):
        xm = xn[:, D * m:D * (m + 1)]
        cm = ct_ref[m]
        ips = jnp.dot(xm, cm, preferred_element_type=jnp.float32,
                      precision=lax.Precision.HIGHEST)
        zmax = jnp.max(ips, axis=-1, keepdims=True)
        e = jnp.exp(ips - zmax)
        s = jnp.sum(e, axis=-1, keepdims=True)
        codes_m = e / s
        codes_ref[:, m, :] = codes_m
        xh = jnp.dot(codes_m, ck_ref[m], preferred_element_type=jnp.float32,
                     precision=lax.Precision.HIGHEST)
        xhat_ref[:, D * m:D * (m + 1)] = xh


def kernel(x, C):
    ct = jnp.transpose(C, (0, 2, 1))
    grid = (B // BB,)
    xhat, codes = pl.pallas_call(
        _pq_body,
        grid=grid,
        in_specs=[
            pl.BlockSpec((BB, F), lambda i: (i, 0)),
            pl.BlockSpec((M, D, K), lambda i: (0, 0, 0)),
            pl.BlockSpec((M, K, D), lambda i: (0, 0, 0)),
        ],
        out_specs=[
            pl.BlockSpec((BB, F), lambda i: (i, 0)),
            pl.BlockSpec((BB, M, K), lambda i: (i, 0, 0)),
        ],
        out_shape=[
            jax.ShapeDtypeStruct((B, F), jnp.float32),
            jax.ShapeDtypeStruct((B, M, K), jnp.float32),
        ],
    )(x, ct, C)
    return xhat, codes
